# baseline (speedup 1.0000x reference)
import functools

import jax
import jax.numpy as jnp
from jax import lax
from jax.experimental import pallas as pl
from jax.experimental.pallas import tpu as pltpu
from jax.experimental.pallas import tpu_sc as plsc

_N = 100000
_E = 1600000
_NSUB = 16
_G = 80
_GPC = 5
_C = _G * _GPC
_EPS = _E // _NSUB
_NCH = _EPS // _C

_f32 = jnp.float32


def _sc_edge_pass(first):
    mesh = plsc.VectorSubcoreMesh(core_axis_name="c", subcore_axis_name="s")

    out_type = [jax.ShapeDtypeStruct((_N, 16), _f32),
                jax.ShapeDtypeStruct((_N, 16), _f32)]
    ring = []
    for _ in range(2):
        ring += [pltpu.VMEM((_C,), jnp.int32),
                 pltpu.VMEM((_C,), jnp.int32),
                 pltpu.VMEM((_C,), _f32),
                 pltpu.VMEM((_C,), _f32),
                 pltpu.VMEM((_C,), _f32),
                 pltpu.VMEM((_C, 16), _f32)]
    scratch = [
        pltpu.VMEM_SHARED((_N, 16), _f32),
        *ring,
        pltpu.VMEM((3, 2, 16), _f32),
    ]
    if first:
        scratch += [pltpu.VMEM((200, 16), _f32)]
    else:
        scratch += [pltpu.VMEM((_C, 16), _f32), pltpu.VMEM((_C, 16), _f32)]
    scratch += [
        pltpu.SemaphoreType.DMA,
        pltpu.SemaphoreType.DMA,
        pltpu.SemaphoreType.DMA,
        pltpu.SemaphoreType.DMA,
    ]
    if first:
        out_type.append(jax.ShapeDtypeStruct((_N,), _f32))
        scratch += [
            pltpu.VMEM_SHARED((_N,), _f32),
            pltpu.VMEM((112,), _f32),
            pltpu.VMEM((816,), _f32),
        ]

    def body(src1, dst1, ea0, ea1, ea2, p0, p1, wb, s0, s1, *rest):
        if first:
            cnt_out = rest[0]
            rest = rest[1:]
        S_sh = rest[0]
        srcv = [rest[1], rest[7]]
        dstv = [rest[2], rest[8]]
        eav = [(rest[3], rest[4], rest[5]), (rest[9], rest[10], rest[11])]
        prow = [rest[6], rest[12]]
        wbv = rest[13]
        if first:
            zrows = rest[14]
            orow = prow
            sem_l, sem_d, sem_g, sem_sc = rest[15:19]
            cnt_sh, onesv, stgv = rest[19:]
        else:
            orow = [rest[14], rest[15]]
            zrows = orow[0].at[pl.ds(0, 200)]
            sem_l, sem_d, sem_g, sem_sc = rest[16:20]
        s = lax.axis_index("s")
        c = lax.axis_index("c")

        def rows_split(fn):
            @pl.when(s < _NSUB - 1)
            def _():
                fn(s * 6400, 6400)

            @pl.when(s == _NSUB - 1)
            def _():
                fn(96000, 4000)

        pltpu.sync_copy(wb, wbv)

        def zrb(j, carry):
            zrows[j, :] = jnp.zeros((16,), _f32)
            return carry
        lax.fori_loop(0, 200, zrb, 0)

        def szero(off, n):
            for k in range(n // 200):
                pltpu.sync_copy(zrows, S_sh.at[pl.ds(off + k * 200, 200)])
        rows_split(szero)
        if first:
            @pl.when(c == 0)
            def _():
                def ob(j, carry):
                    onesv[pl.ds(j * 16, 16)] = jnp.full((16,), 1.0, _f32)
                    return carry
                lax.fori_loop(0, 7, ob, 0)

                def zb(j, carry):
                    stgv[pl.ds(j * 16, 16)] = jnp.zeros((16,), _f32)
                    return carry
                lax.fori_loop(0, 51, zb, 0)

                def czero(off, n):
                    for k in range(n // 800):
                        pltpu.sync_copy(stgv.at[pl.ds(0, 800)],
                                        cnt_sh.at[pl.ds(off + k * 800, 800)])
                rows_split(czero)
        plsc.subcore_barrier()

        def half(h, p_hbm, out_hbm):
            w0 = wbv[0, h, :]
            w1 = wbv[1, h, :]
            w2 = wbv[2, h, :]
            do_cnt = first and h == 0

            def issue_se(u, b):
                off = s * _EPS + u * _C
                pltpu.async_copy(src1.at[pl.ds(off, _C)], srcv[b], sem_l)
                pltpu.async_copy(ea0.at[pl.ds(off, _C)], eav[b][0], sem_l)
                pltpu.async_copy(ea1.at[pl.ds(off, _C)], eav[b][1], sem_l)
                pltpu.async_copy(ea2.at[pl.ds(off, _C)], eav[b][2], sem_l)

            def drain_se(b):
                pltpu.make_async_copy(p_hbm.at[pl.ds(0, 100)],
                                      prow[b].at[pl.ds(0, 100)], sem_l).wait()

            def issue_d(u, b):
                pltpu.async_copy(dst1.at[pl.ds(s * _EPS + u * _C, _C)],
                                 dstv[b], sem_d)

            def drain_d(b):
                pltpu.make_async_copy(dst1.at[pl.ds(0, _C)],
                                      dstv[b], sem_d).wait()

            def issue_gathers(b):
                for j in range(_GPC):
                    pltpu.async_copy(p_hbm.at[srcv[b].at[pl.ds(j * _G, _G)]],
                                     prow[b].at[pl.ds(j * _G, _G)], sem_g)

            def drain_gathers(b):
                pltpu.make_async_copy(p_hbm.at[pl.ds(0, _C)],
                                      prow[b], sem_g).wait()

            def issue_scatters(b):
                for j in range(_GPC):
                    pltpu.async_copy(orow[b].at[pl.ds(j * _G, _G)],
                                     S_sh.at[dstv[b].at[pl.ds(j * _G, _G)]],
                                     sem_sc, add=True)
                    if do_cnt:
                        pltpu.async_copy(onesv.at[pl.ds(0, _G)],
                                         cnt_sh.at[dstv[b].at[pl.ds(j * _G, _G)]],
                                         sem_sc, add=True)

            def drain_scatters(b):
                pltpu.make_async_copy(orow[b],
                                      S_sh.at[pl.ds(0, _C)], sem_sc).wait()
                if do_cnt:
                    pltpu.make_async_copy(stgv.at[pl.ds(0, _C)],
                                          cnt_sh.at[pl.ds(0, _C)],
                                          sem_sc).wait()

            def compute(b):
                pr = prow[b]
                orw = orow[b]
                e0v, e1v, e2v = eav[b]

                def grp(ii, carry3):
                    base = ii * 16
                    v0 = e0v[pl.ds(base, 16)]
                    v1 = e1v[pl.ds(base, 16)]
                    v2 = e2v[pl.ds(base, 16)]
                    for r in range(16):
                        a = v0[r] * w0 + v1[r] * w1 + v2[r] * w2
                        i = base + r
                        orw[i, :] = jnp.maximum(pr[i, :] + a, 0.0)
                    return carry3
                lax.fori_loop(0, _C // 16, grp, 0, unroll=2)

            issue_se(0, 0)
            issue_se(1, 1)
            drain_se(0)
            issue_gathers(0)
            if first:
                issue_d(0, 0)

                def pair(tt, carry):
                    for b in range(2):
                        t = 2 * tt + b
                        nb = 1 - b

                        @pl.when(t >= 1)
                        def _():
                            drain_scatters(nb)

                        @pl.when(t + 1 < _NCH)
                        def _():
                            drain_se(nb)
                            issue_gathers(nb)
                            issue_d(t + 1, nb)

                        drain_gathers(b)
                        compute(b)

                        @pl.when(t + 2 < _NCH)
                        def _():
                            issue_se(t + 2, b)

                        drain_d(b)
                        issue_scatters(b)
                    return carry
                lax.fori_loop(0, _NCH // 2, pair, 0)
                drain_scatters((_NCH - 1) % 2)
            else:
                def pair(tt, carry):
                    for b in range(2):
                        t = 2 * tt + b
                        nb = 1 - b

                        @pl.when(t + 1 < _NCH)
                        def _():
                            drain_se(nb)

                        @pl.when(t >= 2)
                        def _():
                            drain_scatters(b)

                        issue_d(t, b)

                        @pl.when(t + 1 < _NCH)
                        def _():
                            issue_gathers(nb)

                        drain_gathers(b)
                        compute(b)

                        @pl.when(t + 2 < _NCH)
                        def _():
                            issue_se(t + 2, b)

                        drain_d(b)
                        issue_scatters(b)
                    return carry
                lax.fori_loop(0, _NCH // 2, pair, 0)
                drain_scatters(_NCH % 2)
                drain_scatters((_NCH - 1) % 2)

            plsc.subcore_barrier()

            def scopy(off, n):
                for k in range(n // 200):
                    pltpu.sync_copy(S_sh.at[pl.ds(off + k * 200, 200)], zrows)
                    pltpu.sync_copy(zrows,
                                    out_hbm.at[pl.ds(off + k * 200, 200)])
            rows_split(scopy)
            if first and h == 0:
                def cout(off, n):
                    for k in range(n // 800):
                        pltpu.sync_copy(cnt_sh.at[pl.ds(off + k * 800, 800)],
                                        stgv.at[pl.ds(0, 800)])
                        pltpu.sync_copy(stgv.at[pl.ds(0, 800)],
                                        cnt_out.at[pl.ds(off + k * 800, 800)])
                rows_split(cout)

        @pl.when(c == 0)
        def _():
            half(0, p0, s0)

        @pl.when(c == 1)
        def _():
            half(1, p1, s1)

    return pl.kernel(body, out_type=out_type, mesh=mesh, scratch_types=scratch,
                     compiler_params=pltpu.CompilerParams(
                         use_tc_tiling_on_sc=False))


_BN = 2000
_EB = 12800


def _split_ea(edge_attr):
    def tc_body(ea_ref, o0, o1, o2):
        i = pl.program_id(0)
        blk = ea_ref[...]
        t = jax.lax.dot_general(jnp.eye(3, dtype=_f32), blk,
                                (((1,), (1,)), ((), ())),
                                preferred_element_type=_f32)
        o0[pl.ds(i * _EB, _EB)] = t[0, :]
        o1[pl.ds(i * _EB, _EB)] = t[1, :]
        o2[pl.ds(i * _EB, _EB)] = t[2, :]

    return pl.pallas_call(
        tc_body,
        grid=(_E // _EB,),
        in_specs=[pl.BlockSpec((_EB, 3), lambda i: (i, 0))],
        out_specs=[pl.BlockSpec((_E,), lambda i: (0,))] * 3,
        out_shape=[jax.ShapeDtypeStruct((_E,), _f32)] * 3,
    )(edge_attr)


def _split_edges(edge_index):
    def tc_body(ei_ref, src_ref, dst_ref):
        src_ref[...] = ei_ref[0, :]
        dst_ref[...] = ei_ref[1, :]

    return pl.pallas_call(
        tc_body,
        grid=(1,),
        in_specs=[pl.BlockSpec((2, _E), lambda i: (0, 0))],
        out_specs=[pl.BlockSpec((_E,), lambda i: (0,)),
                   pl.BlockSpec((_E,), lambda i: (0,))],
        out_shape=[jax.ShapeDtypeStruct((_E,), jnp.int32)] * 2,
    )(edge_index)


def _proj_first(x, W, b):
    def tc_body(x_ref, w_ref, b_ref, o0_ref, o1_ref):
        h = jnp.dot(x_ref[...], w_ref[...],
                    preferred_element_type=_f32) + b_ref[...]
        o0_ref[...] = h[:, :16]
        o1_ref[...] = h[:, 16:]

    return pl.pallas_call(
        tc_body,
        grid=(_N // _BN,),
        in_specs=[pl.BlockSpec((_BN, 6), lambda i: (i, 0)),
                  pl.BlockSpec((6, 32), lambda i: (0, 0)),
                  pl.BlockSpec((1, 32), lambda i: (0, 0))],
        out_specs=[pl.BlockSpec((_BN, 16), lambda i: (i, 0)),
                   pl.BlockSpec((_BN, 16), lambda i: (i, 0))],
        out_shape=[jax.ShapeDtypeStruct((_N, 16), _f32)] * 2,
    )(x, W, b.reshape(1, 32))


def _mid(s0, s1, cnt, W2, b2, W1n, b1n):
    def tc_body(s0_ref, s1_ref, c_ref, w2_ref, b2_ref, w1_ref, b1_ref,
                o0_ref, o1_ref):
        S = jnp.concatenate([s0_ref[...], s1_ref[...]], axis=1)
        cnt_b = c_ref[...]
        inv = 1.0 / jnp.maximum(cnt_b, 1.0)
        mask = jnp.where(cnt_b > 0.0, 1.0, 0.0)
        m = S * inv
        hh = jnp.maximum(jnp.dot(m, w2_ref[...], preferred_element_type=_f32)
                         + mask * b2_ref[...], 0.0)
        p = jnp.dot(hh, w1_ref[...], preferred_element_type=_f32) + b1_ref[...]
        o0_ref[...] = p[:, :16]
        o1_ref[...] = p[:, 16:]

    return pl.pallas_call(
        tc_body,
        grid=(_N // _BN,),
        in_specs=[pl.BlockSpec((_BN, 16), lambda i: (i, 0)),
                  pl.BlockSpec((_BN, 16), lambda i: (i, 0)),
                  pl.BlockSpec((_BN, 1), lambda i: (i, 0)),
                  pl.BlockSpec((32, 32), lambda i: (0, 0)),
                  pl.BlockSpec((1, 32), lambda i: (0, 0)),
                  pl.BlockSpec((32, 32), lambda i: (0, 0)),
                  pl.BlockSpec((1, 32), lambda i: (0, 0))],
        out_specs=[pl.BlockSpec((_BN, 16), lambda i: (i, 0)),
                   pl.BlockSpec((_BN, 16), lambda i: (i, 0))],
        out_shape=[jax.ShapeDtypeStruct((_N, 16), _f32)] * 2,
    )(s0, s1, cnt.reshape(_N, 1), W2, b2.reshape(1, 32),
      W1n, b1n.reshape(1, 32))


def _final(s0, s1, cnt, W2, b2):
    def tc_body(s0_ref, s1_ref, c_ref, w2_ref, b2_ref, o_ref):
        S = jnp.concatenate([s0_ref[...], s1_ref[...]], axis=1)
        cnt_b = c_ref[...]
        inv = 1.0 / jnp.maximum(cnt_b, 1.0)
        mask = jnp.where(cnt_b > 0.0, 1.0, 0.0)
        o_ref[...] = (jnp.dot(S * inv, w2_ref[...],
                              preferred_element_type=_f32)
                      + mask * b2_ref[...])

    return pl.pallas_call(
        tc_body,
        grid=(_N // _BN,),
        in_specs=[pl.BlockSpec((_BN, 16), lambda i: (i, 0)),
                  pl.BlockSpec((_BN, 16), lambda i: (i, 0)),
                  pl.BlockSpec((_BN, 1), lambda i: (i, 0)),
                  pl.BlockSpec((32, 32), lambda i: (0, 0)),
                  pl.BlockSpec((1, 32), lambda i: (0, 0))],
        out_specs=pl.BlockSpec((_BN, 32), lambda i: (i, 0)),
        out_shape=jax.ShapeDtypeStruct((_N, 32), _f32),
    )(s0, s1, cnt.reshape(_N, 1), W2, b2.reshape(1, 32))


def kernel(x, edge_index, edge_attr,
           l1_W1, l1_b1, l1_W2, l1_b2,
           l2_W1, l2_b1, l2_W2, l2_b2,
           l3_W1, l3_b1, l3_W2, l3_b2):
    src1, dst1 = _split_edges(edge_index)
    ea0, ea1, ea2 = _split_ea(edge_attr * jnp.float32(1.0))
    wb1 = l1_W1[6:].reshape(3, 2, 16)
    wb2 = l2_W1[32:].reshape(3, 2, 16)
    wb3 = l3_W1[32:].reshape(3, 2, 16)

    sc_first = _sc_edge_pass(first=True)
    sc_rest = _sc_edge_pass(first=False)

    p0, p1 = _proj_first(x, l1_W1[:6], l1_b1)
    s0, s1, cnt = sc_first(src1, dst1, ea0, ea1, ea2, p0, p1, wb1)
    p0, p1 = _mid(s0, s1, cnt, l1_W2, l1_b2, l2_W1[:32], l2_b1)
    s0, s1 = sc_rest(src1, dst1, ea0, ea1, ea2, p0, p1, wb2)
    p0, p1 = _mid(s0, s1, cnt, l2_W2, l2_b2, l3_W1[:32], l3_b1)
    s0, s1 = sc_rest(src1, dst1, ea0, ea1, ea2, p0, p1, wb3)
    return _final(s0, s1, cnt, l3_W2, l3_b2)


# speedup vs baseline: 1.0792x; 1.0792x over previous
import functools

import jax
import jax.numpy as jnp
from jax import lax
from jax.experimental import pallas as pl
from jax.experimental.pallas import tpu as pltpu
from jax.experimental.pallas import tpu_sc as plsc

_N = 100000
_E = 1600000
_NSUB = 16
_G = 80
_GPC = 5
_C = _G * _GPC
_EPS = _E // _NSUB
_NCH = _EPS // _C

_f32 = jnp.float32


def _sc_edge_pass(first):
    mesh = plsc.VectorSubcoreMesh(core_axis_name="c", subcore_axis_name="s")

    out_type = [jax.ShapeDtypeStruct((_N, 16), _f32),
                jax.ShapeDtypeStruct((_N, 16), _f32)]
    ring = []
    for _ in range(2):
        ring += [pltpu.VMEM((_C,), jnp.int32),
                 pltpu.VMEM((_C,), jnp.int32),
                 pltpu.VMEM((_C,), _f32),
                 pltpu.VMEM((_C,), _f32),
                 pltpu.VMEM((_C,), _f32),
                 pltpu.VMEM((_C, 16), _f32)]
    scratch = [
        pltpu.VMEM_SHARED((_N, 16), _f32),
        *ring,
        pltpu.VMEM((3, 2, 16), _f32),
        pltpu.VMEM((200, 16), _f32),
        pltpu.SemaphoreType.DMA,
        pltpu.SemaphoreType.DMA,
        pltpu.SemaphoreType.DMA,
        pltpu.SemaphoreType.DMA,
    ]
    if first:
        out_type.append(jax.ShapeDtypeStruct((_N,), _f32))
        scratch += [
            pltpu.VMEM_SHARED((_N,), _f32),
            pltpu.VMEM((112,), _f32),
            pltpu.VMEM((816,), _f32),
        ]

    def body(src1, dst1, ea0, ea1, ea2, p0, p1, wb, s0, s1, *rest):
        if first:
            cnt_out = rest[0]
            rest = rest[1:]
        S_sh = rest[0]
        srcv = [rest[1], rest[7]]
        dstv = [rest[2], rest[8]]
        eav = [(rest[3], rest[4], rest[5]), (rest[9], rest[10], rest[11])]
        prow = [rest[6], rest[12]]
        wbv, zrows, sem_l, sem_d, sem_g, sem_sc = rest[13:19]
        orow = prow
        if first:
            cnt_sh, onesv, stgv = rest[19:]
        s = lax.axis_index("s")
        c = lax.axis_index("c")

        def rows_split(fn):
            @pl.when(s < _NSUB - 1)
            def _():
                fn(s * 6400, 6400)

            @pl.when(s == _NSUB - 1)
            def _():
                fn(96000, 4000)

        pltpu.sync_copy(wb, wbv)

        def zrb(j, carry):
            zrows[j, :] = jnp.zeros((16,), _f32)
            return carry
        lax.fori_loop(0, 200, zrb, 0)

        def szero(off, n):
            for k in range(n // 200):
                pltpu.sync_copy(zrows, S_sh.at[pl.ds(off + k * 200, 200)])
        rows_split(szero)
        if first:
            @pl.when(c == 0)
            def _():
                def ob(j, carry):
                    onesv[pl.ds(j * 16, 16)] = jnp.full((16,), 1.0, _f32)
                    return carry
                lax.fori_loop(0, 7, ob, 0)

                def zb(j, carry):
                    stgv[pl.ds(j * 16, 16)] = jnp.zeros((16,), _f32)
                    return carry
                lax.fori_loop(0, 51, zb, 0)

                def czero(off, n):
                    for k in range(n // 800):
                        pltpu.sync_copy(stgv.at[pl.ds(0, 800)],
                                        cnt_sh.at[pl.ds(off + k * 800, 800)])
                rows_split(czero)
        plsc.subcore_barrier()

        def half(h, p_hbm, out_hbm):
            w0 = wbv[0, h, :]
            w1 = wbv[1, h, :]
            w2 = wbv[2, h, :]
            do_cnt = first and h == 0

            def issue_se(u, b):
                off = s * _EPS + u * _C
                pltpu.async_copy(src1.at[pl.ds(off, _C)], srcv[b], sem_l)
                pltpu.async_copy(ea0.at[pl.ds(off, _C)], eav[b][0], sem_l)
                pltpu.async_copy(ea1.at[pl.ds(off, _C)], eav[b][1], sem_l)
                pltpu.async_copy(ea2.at[pl.ds(off, _C)], eav[b][2], sem_l)

            def drain_se(b):
                pltpu.make_async_copy(p_hbm.at[pl.ds(0, 100)],
                                      prow[b].at[pl.ds(0, 100)], sem_l).wait()

            def issue_d(u, b):
                pltpu.async_copy(dst1.at[pl.ds(s * _EPS + u * _C, _C)],
                                 dstv[b], sem_d)

            def drain_d(b):
                pltpu.make_async_copy(dst1.at[pl.ds(0, _C)],
                                      dstv[b], sem_d).wait()

            def issue_gathers(b):
                for j in range(_GPC):
                    pltpu.async_copy(p_hbm.at[srcv[b].at[pl.ds(j * _G, _G)]],
                                     prow[b].at[pl.ds(j * _G, _G)], sem_g)

            def drain_gathers(b):
                pltpu.make_async_copy(p_hbm.at[pl.ds(0, _C)],
                                      prow[b], sem_g).wait()

            def issue_scatters(b):
                for j in range(_GPC):
                    pltpu.async_copy(orow[b].at[pl.ds(j * _G, _G)],
                                     S_sh.at[dstv[b].at[pl.ds(j * _G, _G)]],
                                     sem_sc, add=True)
                    if do_cnt:
                        pltpu.async_copy(onesv.at[pl.ds(0, _G)],
                                         cnt_sh.at[dstv[b].at[pl.ds(j * _G, _G)]],
                                         sem_sc, add=True)

            def drain_scatters(b):
                pltpu.make_async_copy(orow[b],
                                      S_sh.at[pl.ds(0, _C)], sem_sc).wait()
                if do_cnt:
                    pltpu.make_async_copy(stgv.at[pl.ds(0, _C)],
                                          cnt_sh.at[pl.ds(0, _C)],
                                          sem_sc).wait()

            def compute(b):
                pr = prow[b]
                orw = orow[b]
                e0v, e1v, e2v = eav[b]

                def grp(ii, carry3):
                    base = ii * 16
                    v0 = e0v[pl.ds(base, 16)]
                    v1 = e1v[pl.ds(base, 16)]
                    v2 = e2v[pl.ds(base, 16)]
                    for r in range(16):
                        a = v0[r] * w0 + v1[r] * w1 + v2[r] * w2
                        i = base + r
                        orw[i, :] = jnp.maximum(pr[i, :] + a, 0.0)
                    return carry3
                lax.fori_loop(0, _C // 16, grp, 0, unroll=2)

            issue_se(0, 0)
            issue_se(1, 1)
            drain_se(0)
            issue_gathers(0)
            issue_d(0, 0)

            def pair(tt, carry):
                for b in range(2):
                    t = 2 * tt + b
                    nb = 1 - b

                    @pl.when(t >= 1)
                    def _():
                        drain_scatters(nb)

                    @pl.when(t + 1 < _NCH)
                    def _():
                        drain_se(nb)
                        issue_gathers(nb)
                        issue_d(t + 1, nb)

                    drain_gathers(b)
                    compute(b)

                    @pl.when(t + 2 < _NCH)
                    def _():
                        issue_se(t + 2, b)

                    drain_d(b)
                    issue_scatters(b)
                return carry
            lax.fori_loop(0, _NCH // 2, pair, 0)
            drain_scatters((_NCH - 1) % 2)

            plsc.subcore_barrier()

            def scopy(off, n):
                for k in range(n // 200):
                    pltpu.sync_copy(S_sh.at[pl.ds(off + k * 200, 200)], zrows)
                    pltpu.sync_copy(zrows,
                                    out_hbm.at[pl.ds(off + k * 200, 200)])
            rows_split(scopy)
            if first and h == 0:
                def cout(off, n):
                    for k in range(n // 800):
                        pltpu.sync_copy(cnt_sh.at[pl.ds(off + k * 800, 800)],
                                        stgv.at[pl.ds(0, 800)])
                        pltpu.sync_copy(stgv.at[pl.ds(0, 800)],
                                        cnt_out.at[pl.ds(off + k * 800, 800)])
                rows_split(cout)

        @pl.when(c == 0)
        def _():
            half(0, p0, s0)

        @pl.when(c == 1)
        def _():
            half(1, p1, s1)

    return pl.kernel(body, out_type=out_type, mesh=mesh, scratch_types=scratch,
                     compiler_params=pltpu.CompilerParams(
                         use_tc_tiling_on_sc=False))


_BN = 2000
_EB = 12800


def _split_ea(edge_attr):
    def tc_body(ea_ref, o0, o1, o2):
        i = pl.program_id(0)
        blk = ea_ref[...]
        t = jax.lax.dot_general(jnp.eye(3, dtype=_f32), blk,
                                (((1,), (1,)), ((), ())),
                                preferred_element_type=_f32)
        o0[pl.ds(i * _EB, _EB)] = t[0, :]
        o1[pl.ds(i * _EB, _EB)] = t[1, :]
        o2[pl.ds(i * _EB, _EB)] = t[2, :]

    return pl.pallas_call(
        tc_body,
        grid=(_E // _EB,),
        in_specs=[pl.BlockSpec((_EB, 3), lambda i: (i, 0))],
        out_specs=[pl.BlockSpec((_E,), lambda i: (0,))] * 3,
        out_shape=[jax.ShapeDtypeStruct((_E,), _f32)] * 3,
    )(edge_attr)


def _split_edges(edge_index):
    def tc_body(ei_ref, src_ref, dst_ref):
        src_ref[...] = ei_ref[0, :]
        dst_ref[...] = ei_ref[1, :]

    return pl.pallas_call(
        tc_body,
        grid=(1,),
        in_specs=[pl.BlockSpec((2, _E), lambda i: (0, 0))],
        out_specs=[pl.BlockSpec((_E,), lambda i: (0,)),
                   pl.BlockSpec((_E,), lambda i: (0,))],
        out_shape=[jax.ShapeDtypeStruct((_E,), jnp.int32)] * 2,
    )(edge_index)


def _proj_first(x, W, b):
    def tc_body(x_ref, w_ref, b_ref, o0_ref, o1_ref):
        h = jnp.dot(x_ref[...], w_ref[...],
                    preferred_element_type=_f32) + b_ref[...]
        o0_ref[...] = h[:, :16]
        o1_ref[...] = h[:, 16:]

    return pl.pallas_call(
        tc_body,
        grid=(_N // _BN,),
        in_specs=[pl.BlockSpec((_BN, 6), lambda i: (i, 0)),
                  pl.BlockSpec((6, 32), lambda i: (0, 0)),
                  pl.BlockSpec((1, 32), lambda i: (0, 0))],
        out_specs=[pl.BlockSpec((_BN, 16), lambda i: (i, 0)),
                   pl.BlockSpec((_BN, 16), lambda i: (i, 0))],
        out_shape=[jax.ShapeDtypeStruct((_N, 16), _f32)] * 2,
    )(x, W, b.reshape(1, 32))


def _mid(s0, s1, cnt, W2, b2, W1n, b1n):
    def tc_body(s0_ref, s1_ref, c_ref, w2_ref, b2_ref, w1_ref, b1_ref,
                o0_ref, o1_ref):
        S = jnp.concatenate([s0_ref[...], s1_ref[...]], axis=1)
        cnt_b = c_ref[...]
        inv = 1.0 / jnp.maximum(cnt_b, 1.0)
        mask = jnp.where(cnt_b > 0.0, 1.0, 0.0)
        m = S * inv
        hh = jnp.maximum(jnp.dot(m, w2_ref[...], preferred_element_type=_f32)
                         + mask * b2_ref[...], 0.0)
        p = jnp.dot(hh, w1_ref[...], preferred_element_type=_f32) + b1_ref[...]
        o0_ref[...] = p[:, :16]
        o1_ref[...] = p[:, 16:]

    return pl.pallas_call(
        tc_body,
        grid=(_N // _BN,),
        in_specs=[pl.BlockSpec((_BN, 16), lambda i: (i, 0)),
                  pl.BlockSpec((_BN, 16), lambda i: (i, 0)),
                  pl.BlockSpec((_BN, 1), lambda i: (i, 0)),
                  pl.BlockSpec((32, 32), lambda i: (0, 0)),
                  pl.BlockSpec((1, 32), lambda i: (0, 0)),
                  pl.BlockSpec((32, 32), lambda i: (0, 0)),
                  pl.BlockSpec((1, 32), lambda i: (0, 0))],
        out_specs=[pl.BlockSpec((_BN, 16), lambda i: (i, 0)),
                   pl.BlockSpec((_BN, 16), lambda i: (i, 0))],
        out_shape=[jax.ShapeDtypeStruct((_N, 16), _f32)] * 2,
    )(s0, s1, cnt.reshape(_N, 1), W2, b2.reshape(1, 32),
      W1n, b1n.reshape(1, 32))


def _final(s0, s1, cnt, W2, b2):
    def tc_body(s0_ref, s1_ref, c_ref, w2_ref, b2_ref, o_ref):
        S = jnp.concatenate([s0_ref[...], s1_ref[...]], axis=1)
        cnt_b = c_ref[...]
        inv = 1.0 / jnp.maximum(cnt_b, 1.0)
        mask = jnp.where(cnt_b > 0.0, 1.0, 0.0)
        o_ref[...] = (jnp.dot(S * inv, w2_ref[...],
                              preferred_element_type=_f32)
                      + mask * b2_ref[...])

    return pl.pallas_call(
        tc_body,
        grid=(_N // _BN,),
        in_specs=[pl.BlockSpec((_BN, 16), lambda i: (i, 0)),
                  pl.BlockSpec((_BN, 16), lambda i: (i, 0)),
                  pl.BlockSpec((_BN, 1), lambda i: (i, 0)),
                  pl.BlockSpec((32, 32), lambda i: (0, 0)),
                  pl.BlockSpec((1, 32), lambda i: (0, 0))],
        out_specs=pl.BlockSpec((_BN, 32), lambda i: (i, 0)),
        out_shape=jax.ShapeDtypeStruct((_N, 32), _f32),
    )(s0, s1, cnt.reshape(_N, 1), W2, b2.reshape(1, 32))


def kernel(x, edge_index, edge_attr,
           l1_W1, l1_b1, l1_W2, l1_b2,
           l2_W1, l2_b1, l2_W2, l2_b2,
           l3_W1, l3_b1, l3_W2, l3_b2):
    src1, dst1 = _split_edges(edge_index)
    ea0, ea1, ea2 = _split_ea(edge_attr * jnp.float32(1.0))
    wb1 = l1_W1[6:].reshape(3, 2, 16)
    wb2 = l2_W1[32:].reshape(3, 2, 16)
    wb3 = l3_W1[32:].reshape(3, 2, 16)

    sc_first = _sc_edge_pass(first=True)
    sc_rest = _sc_edge_pass(first=False)

    p0, p1 = _proj_first(x, l1_W1[:6], l1_b1)
    s0, s1, cnt = sc_first(src1, dst1, ea0, ea1, ea2, p0, p1, wb1)
    p0, p1 = _mid(s0, s1, cnt, l1_W2, l1_b2, l2_W1[:32], l2_b1)
    s0, s1 = sc_rest(src1, dst1, ea0, ea1, ea2, p0, p1, wb2)
    p0, p1 = _mid(s0, s1, cnt, l2_W2, l2_b2, l3_W1[:32], l3_b1)
    s0, s1 = sc_rest(src1, dst1, ea0, ea1, ea2, p0, p1, wb3)
    return _final(s0, s1, cnt, l3_W2, l3_b2)
